# Initial kernel scaffold; baseline (speedup 1.0000x reference)
import jax
import jax.numpy as jnp
from jax.experimental import pallas as pl


def kernel(input, permanences, duty_cycles):
    raise NotImplementedError("write your pallas kernel here")


# speedup vs baseline: 4.1915x; 4.1915x over previous
import functools

import numpy as np

import jax
import jax.numpy as jnp
from jax import lax
from jax.experimental import pallas as pl
from jax.experimental.pallas import tpu as pltpu
from jax.experimental.pallas import tpu_sc as plsc

INPUT_DIM = 4096
COLUMN_DIM = 16384
SPARSITY = 0.02
BATCH = 256
K_TOP = int(round(SPARSITY * COLUMN_DIM))
BOOST_BETA = 100.0

BN = 1024
R2 = 64
CAND = 512
RANKW = 384
R4 = 8

_INTERPRET = False


def _matmul_body(x_ref, p_ref, b_ref, ov_ref, bo_ref):
    x = x_ref[...]
    p = p_ref[...]
    ov = jax.lax.dot_general(x, p, (((1,), (1,)), ((), ())),
                             preferred_element_type=jnp.float32)
    ov_ref[...] = ov
    bo_ref[...] = ov * b_ref[...]


def _overlaps_boosted(inp, perm, boost2d):
    return pl.pallas_call(
        _matmul_body,
        grid=(COLUMN_DIM // BN,),
        in_specs=[
            pl.BlockSpec((BATCH, INPUT_DIM), lambda j: (0, 0)),
            pl.BlockSpec((BN, INPUT_DIM), lambda j: (j, 0)),
            pl.BlockSpec((1, BN), lambda j: (0, j)),
        ],
        out_specs=[
            pl.BlockSpec((BATCH, BN), lambda j: (0, j)),
            pl.BlockSpec((BATCH, BN), lambda j: (0, j)),
        ],
        out_shape=[
            jax.ShapeDtypeStruct((BATCH, COLUMN_DIM), jnp.float32),
            jax.ShapeDtypeStruct((BATCH, COLUMN_DIM), jnp.float32),
        ],
        interpret=_INTERPRET,
    )(inp, perm, boost2d)


def _threshold_body(bo_ref, bits_ref, thr_ref, need_ref):
    v = bo_ref[...]
    vi = jax.lax.bitcast_convert_type(v, jnp.int32)

    def step(_, carry):
        lo, hi = carry
        mid = lo + jax.lax.div(hi - lo + 1, 2)
        cnt = jnp.sum((vi >= mid).astype(jnp.int32), axis=1, keepdims=True)
        pred = cnt >= K_TOP
        return jnp.where(pred, mid, lo), jnp.where(pred, hi, mid - 1)

    lo0 = jnp.zeros((R2, 1), jnp.int32)
    hi0 = jnp.full((R2, 1), int(np.float32(30266.0).view(np.int32)), jnp.int32)
    lo, _ = jax.lax.fori_loop(0, 31, step, (lo0, hi0))
    thr_i = lo
    gt = (vi > thr_i)
    count_gt = jnp.sum(gt.astype(jnp.int32), axis=1, keepdims=True)
    need = K_TOP - count_gt

    tie = (vi == thr_i).astype(jnp.float32)
    t3 = tie.reshape(R2 * 128, 128)
    ltri = (jax.lax.broadcasted_iota(jnp.int32, (128, 128), 0)
            < jax.lax.broadcasted_iota(jnp.int32, (128, 128), 1)
            ).astype(jnp.float32)
    inner = jax.lax.dot_general(t3, ltri, (((1,), (0,)), ((), ())),
                                preferred_element_type=jnp.float32)
    chunk_tot = jnp.sum(t3, axis=1).reshape(R2, 128)
    chunk_off = jax.lax.dot_general(chunk_tot, ltri, (((1,), (0,)), ((), ())),
                                    preferred_element_type=jnp.float32)
    tie_rank = (inner.reshape(R2, 128, 128)
                + chunk_off[:, :, None]).reshape(R2, COLUMN_DIM)

    winner = gt | ((vi == thr_i) & (tie_rank < need.astype(jnp.float32)))
    bits_ref[...] = winner.astype(jnp.float32)
    thr_ref[...] = jax.lax.bitcast_convert_type(thr_i, jnp.float32)
    need_ref[...] = need


def _threshold(boosted):
    return pl.pallas_call(
        _threshold_body,
        grid=(BATCH // R2,),
        in_specs=[pl.BlockSpec((R2, COLUMN_DIM), lambda i: (i, 0))],
        out_specs=[
            pl.BlockSpec((R2, COLUMN_DIM), lambda i: (i, 0)),
            pl.BlockSpec((R2, 1), lambda i: (i, 0)),
            pl.BlockSpec((R2, 1), lambda i: (i, 0)),
        ],
        out_shape=[
            jax.ShapeDtypeStruct((BATCH, COLUMN_DIM), jnp.float32),
            jax.ShapeDtypeStruct((BATCH, 1), jnp.float32),
            jax.ShapeDtypeStruct((BATCH, 1), jnp.int32),
        ],
        interpret=_INTERPRET,
    )(boosted)


_SC_L = 16
_SC_WORKERS = 32


def _sc_compact_body(bits_hbm, bo_hbm, idx_hbm, val_hbm,
                     bits_v, bo_v, ci_v, cv_v):
    L = _SC_L
    rows_per_w = BATCH // _SC_WORKERS
    wid = lax.axis_index("s") * 2 + lax.axis_index("c")
    iota = lax.iota(jnp.int32, L)
    pad_i = jnp.full((L,), COLUMN_DIM, jnp.int32)
    pad_v = jnp.full((L,), -1.0, jnp.float32)
    for rlocal in range(rows_per_w):
        row = wid * rows_per_w + rlocal
        pltpu.sync_copy(bits_hbm.at[row], bits_v)
        pltpu.sync_copy(bo_hbm.at[row], bo_v)
        for j in range(CAND // L):
            ci_v[pl.ds(j * L, L)] = pad_i
            cv_v[pl.ds(j * L, L)] = pad_v

        def body(j, off):
            b = bits_v[pl.ds(j * L, L)]
            v = bo_v[pl.ds(j * L, L)]
            m = b > 0.5
            plsc.store_compressed(ci_v.at[pl.ds(off, L)], iota + j * L, mask=m)
            plsc.store_compressed(cv_v.at[pl.ds(off, L)], v, mask=m)
            return off + jnp.sum(m.astype(jnp.int32))

        lax.fori_loop(0, COLUMN_DIM // L, body, jnp.int32(0))
        pltpu.sync_copy(ci_v, idx_hbm.at[row])
        pltpu.sync_copy(cv_v, val_hbm.at[row])


def _compact(bits, boosted):
    mesh = plsc.VectorSubcoreMesh(core_axis_name="c", subcore_axis_name="s")
    f = pl.kernel(
        _sc_compact_body, mesh=mesh,
        out_type=[
            jax.ShapeDtypeStruct((BATCH, CAND), jnp.int32),
            jax.ShapeDtypeStruct((BATCH, CAND), jnp.float32),
        ],
        scratch_types=[
            pltpu.VMEM((COLUMN_DIM,), jnp.float32),
            pltpu.VMEM((COLUMN_DIM,), jnp.float32),
            pltpu.VMEM((CAND,), jnp.int32),
            pltpu.VMEM((CAND,), jnp.float32),
        ],
        compiler_params=pltpu.CompilerParams(needs_layout_passes=False),
    )
    return f(bits, boosted)


def _rank_body(ci_ref, cv_ref, out_ref):
    ci = ci_ref[...]
    cv = cv_ref[...]
    va = cv[:, :, None]
    vb = cv[:, None, :]
    ia = ci[:, :, None]
    ib = ci[:, None, :]
    beats = (vb > va) | ((vb == va) & (ib < ia))
    rank = jnp.sum(beats.astype(jnp.int32), axis=2)
    onehot = (rank[:, :, None]
              == jax.lax.broadcasted_iota(jnp.int32, (1, 1, RANKW), 2)
              ).astype(jnp.float32)
    outf = jax.lax.dot_general(ci.astype(jnp.float32), onehot,
                               (((1,), (1,)), ((0,), (0,))),
                               preferred_element_type=jnp.float32)
    out_ref[...] = outf.astype(jnp.int32)


def _rank_scatter(cand_idx, cand_val):
    return pl.pallas_call(
        _rank_body,
        grid=(BATCH // R4,),
        in_specs=[
            pl.BlockSpec((R4, CAND), lambda i: (i, 0)),
            pl.BlockSpec((R4, CAND), lambda i: (i, 0)),
        ],
        out_specs=pl.BlockSpec((R4, RANKW), lambda i: (i, 0)),
        out_shape=jax.ShapeDtypeStruct((BATCH, RANKW), jnp.int32),
        interpret=_INTERPRET,
    )(cand_idx, cand_val)


def kernel(input, permanences, duty_cycles):
    boost = jnp.exp(BOOST_BETA * (SPARSITY - duty_cycles))
    overlaps, boosted = _overlaps_boosted(input, permanences,
                                          boost.reshape(1, COLUMN_DIM))
    bits, _thr, _need = _threshold(boosted)
    cand_idx, cand_val = _compact(bits, boosted)
    active_columns = _rank_scatter(cand_idx, cand_val)[:, :K_TOP]
    return active_columns, overlaps, boosted, bits


# speedup vs baseline: 4.2412x; 1.0119x over previous
import functools

import numpy as np

import jax
import jax.numpy as jnp
from jax import lax
from jax.experimental import pallas as pl
from jax.experimental.pallas import tpu as pltpu
from jax.experimental.pallas import tpu_sc as plsc

INPUT_DIM = 4096
COLUMN_DIM = 16384
SPARSITY = 0.02
BATCH = 256
K_TOP = int(round(SPARSITY * COLUMN_DIM))
BOOST_BETA = 100.0

BN = 1024
R2 = 64
CAND = 512
RANKW = 384
R4 = 8

_INTERPRET = False


def _matmul_body(x_ref, p_ref, b_ref, ov_ref, bo_ref):
    x = x_ref[...]
    p = p_ref[...]
    ov = jax.lax.dot_general(x, p, (((1,), (1,)), ((), ())),
                             preferred_element_type=jnp.float32)
    ov_ref[...] = ov
    bo_ref[...] = ov * b_ref[...]


def _overlaps_boosted(inp, perm, boost2d):
    return pl.pallas_call(
        _matmul_body,
        grid=(COLUMN_DIM // BN,),
        in_specs=[
            pl.BlockSpec((BATCH, INPUT_DIM), lambda j: (0, 0)),
            pl.BlockSpec((BN, INPUT_DIM), lambda j: (j, 0)),
            pl.BlockSpec((1, BN), lambda j: (0, j)),
        ],
        out_specs=[
            pl.BlockSpec((BATCH, BN), lambda j: (0, j)),
            pl.BlockSpec((BATCH, BN), lambda j: (0, j)),
        ],
        out_shape=[
            jax.ShapeDtypeStruct((BATCH, COLUMN_DIM), jnp.float32),
            jax.ShapeDtypeStruct((BATCH, COLUMN_DIM), jnp.float32),
        ],
        interpret=_INTERPRET,
    )(inp, perm, boost2d)


def _excl_prefix(mask_f32, ltri):
    t3 = mask_f32.reshape(R2 * 128, 128)
    inner = jax.lax.dot_general(t3, ltri, (((1,), (0,)), ((), ())),
                                preferred_element_type=jnp.float32)
    chunk_tot = jnp.sum(t3, axis=1).reshape(R2, 128)
    chunk_off = jax.lax.dot_general(chunk_tot, ltri, (((1,), (0,)), ((), ())),
                                    preferred_element_type=jnp.float32)
    return (inner.reshape(R2, 128, 128)
            + chunk_off[:, :, None]).reshape(R2, COLUMN_DIM)


def _threshold_body(bo_ref, bits_ref, slot_ref):
    v = bo_ref[...]
    vi = jax.lax.bitcast_convert_type(v, jnp.int32)

    def step(_, carry):
        lo, hi = carry
        mid = lo + jax.lax.div(hi - lo + 1, 2)
        cnt = jnp.sum((vi >= mid).astype(jnp.int32), axis=1, keepdims=True)
        pred = cnt >= K_TOP
        return jnp.where(pred, mid, lo), jnp.where(pred, hi, mid - 1)

    lo0 = jnp.zeros((R2, 1), jnp.int32)
    hi0 = jnp.full((R2, 1), int(np.float32(30266.0).view(np.int32)), jnp.int32)
    lo, _ = jax.lax.fori_loop(0, 31, step, (lo0, hi0))
    thr_i = lo
    gt = (vi > thr_i)
    count_gt = jnp.sum(gt.astype(jnp.int32), axis=1, keepdims=True)
    need = K_TOP - count_gt

    ltri = (jax.lax.broadcasted_iota(jnp.int32, (128, 128), 0)
            < jax.lax.broadcasted_iota(jnp.int32, (128, 128), 1)
            ).astype(jnp.float32)
    tie_rank = _excl_prefix((vi == thr_i).astype(jnp.float32), ltri)
    winner = gt | ((vi == thr_i) & (tie_rank < need.astype(jnp.float32)))
    bits_ref[...] = winner.astype(jnp.float32)
    wslot = _excl_prefix(winner.astype(jnp.float32), ltri)
    slot_ref[...] = jnp.where(winner, wslot.astype(jnp.int32), CAND - 1)


def _threshold(boosted):
    return pl.pallas_call(
        _threshold_body,
        grid=(BATCH // R2,),
        in_specs=[pl.BlockSpec((R2, COLUMN_DIM), lambda i: (i, 0))],
        out_specs=[
            pl.BlockSpec((R2, COLUMN_DIM), lambda i: (i, 0)),
            pl.BlockSpec((R2, COLUMN_DIM), lambda i: (i, 0)),
        ],
        out_shape=[
            jax.ShapeDtypeStruct((BATCH, COLUMN_DIM), jnp.float32),
            jax.ShapeDtypeStruct((BATCH, COLUMN_DIM), jnp.int32),
        ],
        interpret=_INTERPRET,
    )(boosted)


_SC_L = 16
_SC_WORKERS = 32


def _sc_compact_body(slot_hbm, bo_hbm, idx_hbm, val_hbm,
                     slot_v, bo_v, ci_v, cv_v):
    L = _SC_L
    rows_per_w = BATCH // _SC_WORKERS
    wid = lax.axis_index("s") * 2 + lax.axis_index("c")
    iota = lax.iota(jnp.int32, L)
    pad_i = jnp.full((L,), COLUMN_DIM, jnp.int32)
    pad_v = jnp.full((L,), -1.0, jnp.float32)
    for rlocal in range(rows_per_w):
        row = wid * rows_per_w + rlocal
        pltpu.sync_copy(slot_hbm.at[row], slot_v)
        pltpu.sync_copy(bo_hbm.at[row], bo_v)
        for j in range(CAND // L):
            ci_v[pl.ds(j * L, L)] = pad_i
            cv_v[pl.ds(j * L, L)] = pad_v

        def body(j, c):
            s = slot_v[pl.ds(j * L, L)]
            v = bo_v[pl.ds(j * L, L)]
            m = s < CAND - 1
            plsc.store_scatter(ci_v, [s], iota + j * L, mask=m)
            plsc.store_scatter(cv_v, [s], v, mask=m)
            return c

        lax.fori_loop(0, COLUMN_DIM // L, body, jnp.int32(0))
        pltpu.sync_copy(ci_v, idx_hbm.at[row])
        pltpu.sync_copy(cv_v, val_hbm.at[row])


def _compact(slot, boosted):
    mesh = plsc.VectorSubcoreMesh(core_axis_name="c", subcore_axis_name="s")
    f = pl.kernel(
        _sc_compact_body, mesh=mesh,
        out_type=[
            jax.ShapeDtypeStruct((BATCH, CAND), jnp.int32),
            jax.ShapeDtypeStruct((BATCH, CAND), jnp.float32),
        ],
        scratch_types=[
            pltpu.VMEM((COLUMN_DIM,), jnp.int32),
            pltpu.VMEM((COLUMN_DIM,), jnp.float32),
            pltpu.VMEM((CAND,), jnp.int32),
            pltpu.VMEM((CAND,), jnp.float32),
        ],
        compiler_params=pltpu.CompilerParams(needs_layout_passes=False),
    )
    return f(slot, boosted)


def _rank_body(ci_ref, cv_ref, out_ref):
    ci = ci_ref[...]
    cv = cv_ref[...]
    va = cv[:, :, None]
    vb = cv[:, None, :]
    ia = ci[:, :, None]
    ib = ci[:, None, :]
    beats = (vb > va) | ((vb == va) & (ib < ia))
    rank = jnp.sum(beats.astype(jnp.int32), axis=2)
    onehot = (rank[:, :, None]
              == jax.lax.broadcasted_iota(jnp.int32, (1, 1, RANKW), 2)
              ).astype(jnp.float32)
    outf = jax.lax.dot_general(ci.astype(jnp.float32), onehot,
                               (((1,), (1,)), ((0,), (0,))),
                               preferred_element_type=jnp.float32)
    out_ref[...] = outf.astype(jnp.int32)


def _rank_scatter(cand_idx, cand_val):
    return pl.pallas_call(
        _rank_body,
        grid=(BATCH // R4,),
        in_specs=[
            pl.BlockSpec((R4, CAND), lambda i: (i, 0)),
            pl.BlockSpec((R4, CAND), lambda i: (i, 0)),
        ],
        out_specs=pl.BlockSpec((R4, RANKW), lambda i: (i, 0)),
        out_shape=jax.ShapeDtypeStruct((BATCH, RANKW), jnp.int32),
        interpret=_INTERPRET,
    )(cand_idx, cand_val)


def kernel(input, permanences, duty_cycles):
    boost = jnp.exp(BOOST_BETA * (SPARSITY - duty_cycles))
    overlaps, boosted = _overlaps_boosted(input, permanences,
                                          boost.reshape(1, COLUMN_DIM))
    bits, slot = _threshold(boosted)
    cand_idx, cand_val = _compact(slot, boosted)
    active_columns = _rank_scatter(cand_idx, cand_val)[:, :K_TOP]
    return active_columns, overlaps, boosted, bits


# speedup vs baseline: 4.6766x; 1.1027x over previous
import functools

import numpy as np

import jax
import jax.numpy as jnp
from jax import lax
from jax.experimental import pallas as pl
from jax.experimental.pallas import tpu as pltpu
from jax.experimental.pallas import tpu_sc as plsc

INPUT_DIM = 4096
COLUMN_DIM = 16384
SPARSITY = 0.02
BATCH = 256
K_TOP = int(round(SPARSITY * COLUMN_DIM))
BOOST_BETA = 100.0

BN = 1024
R2 = 64
CAND = 384
RANKW = 384
R4 = 8

_INTERPRET = False


def _matmul_body(x_ref, p_ref, b_ref, ov_ref, bo_ref):
    x = x_ref[...]
    p = p_ref[...]
    ov = jax.lax.dot_general(x, p, (((1,), (1,)), ((), ())),
                             preferred_element_type=jnp.float32)
    ov_ref[...] = ov
    bo_ref[...] = ov * b_ref[...]


def _overlaps_boosted(inp, perm, boost2d):
    return pl.pallas_call(
        _matmul_body,
        grid=(COLUMN_DIM // BN,),
        in_specs=[
            pl.BlockSpec((BATCH, INPUT_DIM), lambda j: (0, 0)),
            pl.BlockSpec((BN, INPUT_DIM), lambda j: (j, 0)),
            pl.BlockSpec((1, BN), lambda j: (0, j)),
        ],
        out_specs=[
            pl.BlockSpec((BATCH, BN), lambda j: (0, j)),
            pl.BlockSpec((BATCH, BN), lambda j: (0, j)),
        ],
        out_shape=[
            jax.ShapeDtypeStruct((BATCH, COLUMN_DIM), jnp.float32),
            jax.ShapeDtypeStruct((BATCH, COLUMN_DIM), jnp.float32),
        ],
        interpret=_INTERPRET,
    )(inp, perm, boost2d)


def _excl_prefix(mask_f32, ltri):
    t3 = mask_f32.reshape(R2 * 128, 128)
    inner = jax.lax.dot_general(t3, ltri, (((1,), (0,)), ((), ())),
                                preferred_element_type=jnp.float32)
    chunk_tot = jnp.sum(t3, axis=1).reshape(R2, 128)
    chunk_off = jax.lax.dot_general(chunk_tot, ltri, (((1,), (0,)), ((), ())),
                                    preferred_element_type=jnp.float32)
    return (inner.reshape(R2, 128, 128)
            + chunk_off[:, :, None]).reshape(R2, COLUMN_DIM)


def _threshold_body(bo_ref, bits_ref, slot_ref):
    v = bo_ref[...]
    vi = jax.lax.bitcast_convert_type(v, jnp.int32)

    def step(_, carry):
        lo, hi = carry
        mid = lo + jax.lax.div(hi - lo + 1, 2)
        cnt = jnp.sum((vi >= mid).astype(jnp.int32), axis=1, keepdims=True)
        pred = cnt >= K_TOP
        return jnp.where(pred, mid, lo), jnp.where(pred, hi, mid - 1)

    lo0 = jnp.zeros((R2, 1), jnp.int32)
    hi0 = jnp.full((R2, 1), int(np.float32(30266.0).view(np.int32)), jnp.int32)
    lo, _ = jax.lax.fori_loop(0, 31, step, (lo0, hi0))
    thr_i = lo
    gt = (vi > thr_i)
    count_gt = jnp.sum(gt.astype(jnp.int32), axis=1, keepdims=True)
    need = K_TOP - count_gt

    ltri = (jax.lax.broadcasted_iota(jnp.int32, (128, 128), 0)
            < jax.lax.broadcasted_iota(jnp.int32, (128, 128), 1)
            ).astype(jnp.float32)
    tie_rank = _excl_prefix((vi == thr_i).astype(jnp.float32), ltri)
    winner = gt | ((vi == thr_i) & (tie_rank < need.astype(jnp.float32)))
    bits_ref[...] = winner.astype(jnp.float32)
    wslot = _excl_prefix(winner.astype(jnp.float32), ltri)
    slot_ref[...] = jnp.where(winner, wslot.astype(jnp.int32), CAND - 1)


def _threshold(boosted):
    return pl.pallas_call(
        _threshold_body,
        grid=(BATCH // R2,),
        in_specs=[pl.BlockSpec((R2, COLUMN_DIM), lambda i: (i, 0))],
        out_specs=[
            pl.BlockSpec((R2, COLUMN_DIM), lambda i: (i, 0)),
            pl.BlockSpec((R2, COLUMN_DIM), lambda i: (i, 0)),
        ],
        out_shape=[
            jax.ShapeDtypeStruct((BATCH, COLUMN_DIM), jnp.float32),
            jax.ShapeDtypeStruct((BATCH, COLUMN_DIM), jnp.int32),
        ],
        interpret=_INTERPRET,
    )(boosted)


_SC_L = 16
_SC_WORKERS = 32


def _sc_compact_body(slot_hbm, bo_hbm, idx_hbm, val_hbm,
                     slot_v, bo_v, ci_v, cv_v):
    L = _SC_L
    rows_per_w = BATCH // _SC_WORKERS
    wid = lax.axis_index("s") * 2 + lax.axis_index("c")
    iota = lax.iota(jnp.int32, L)
    pad_i = jnp.full((L,), COLUMN_DIM, jnp.int32)
    pad_v = jnp.full((L,), -1.0, jnp.float32)
    for rlocal in range(rows_per_w):
        row = wid * rows_per_w + rlocal
        pltpu.sync_copy(slot_hbm.at[row], slot_v)
        pltpu.sync_copy(bo_hbm.at[row], bo_v)
        for j in range(CAND // L):
            ci_v[pl.ds(j * L, L)] = pad_i
            cv_v[pl.ds(j * L, L)] = pad_v

        def body(j, c):
            s = slot_v[pl.ds(j * L, L)]
            v = bo_v[pl.ds(j * L, L)]
            m = s < CAND - 1
            plsc.store_scatter(ci_v, [s], iota + j * L, mask=m)
            plsc.store_scatter(cv_v, [s], v, mask=m)
            return c

        lax.fori_loop(0, COLUMN_DIM // L, body, jnp.int32(0))
        pltpu.sync_copy(ci_v, idx_hbm.at[row])
        pltpu.sync_copy(cv_v, val_hbm.at[row])


def _compact(slot, boosted):
    mesh = plsc.VectorSubcoreMesh(core_axis_name="c", subcore_axis_name="s")
    f = pl.kernel(
        _sc_compact_body, mesh=mesh,
        out_type=[
            jax.ShapeDtypeStruct((BATCH, CAND), jnp.int32),
            jax.ShapeDtypeStruct((BATCH, CAND), jnp.float32),
        ],
        scratch_types=[
            pltpu.VMEM((COLUMN_DIM,), jnp.int32),
            pltpu.VMEM((COLUMN_DIM,), jnp.float32),
            pltpu.VMEM((CAND,), jnp.int32),
            pltpu.VMEM((CAND,), jnp.float32),
        ],
        compiler_params=pltpu.CompilerParams(needs_layout_passes=False),
    )
    return f(slot, boosted)


def _rank_body(ci_ref, cv_ref, out_ref):
    ci = ci_ref[...]
    cv = cv_ref[...]
    va = cv[:, :, None]
    vb = cv[:, None, :]
    ia = ci[:, :, None]
    ib = ci[:, None, :]
    beats = (vb > va) | ((vb == va) & (ib < ia))
    rank = jnp.sum(beats.astype(jnp.int32), axis=2)
    onehot = (rank[:, :, None]
              == jax.lax.broadcasted_iota(jnp.int32, (1, 1, RANKW), 2)
              ).astype(jnp.float32)
    outf = jax.lax.dot_general(ci.astype(jnp.float32), onehot,
                               (((1,), (1,)), ((0,), (0,))),
                               preferred_element_type=jnp.float32)
    out_ref[...] = outf[:, :K_TOP].astype(jnp.int32)


def _rank_scatter(cand_idx, cand_val):
    return pl.pallas_call(
        _rank_body,
        grid=(BATCH // R4,),
        in_specs=[
            pl.BlockSpec((R4, CAND), lambda i: (i, 0)),
            pl.BlockSpec((R4, CAND), lambda i: (i, 0)),
        ],
        out_specs=pl.BlockSpec((R4, K_TOP), lambda i: (i, 0)),
        out_shape=jax.ShapeDtypeStruct((BATCH, K_TOP), jnp.int32),
        interpret=_INTERPRET,
    )(cand_idx, cand_val)


def kernel(input, permanences, duty_cycles):
    boost = jnp.exp(BOOST_BETA * (SPARSITY - duty_cycles))
    overlaps, boosted = _overlaps_boosted(input, permanences,
                                          boost.reshape(1, COLUMN_DIM))
    bits, slot = _threshold(boosted)
    cand_idx, cand_val = _compact(slot, boosted)
    active_columns = _rank_scatter(cand_idx, cand_val)
    return active_columns, overlaps, boosted, bits


# speedup vs baseline: 4.8542x; 1.0380x over previous
import functools

import numpy as np

import jax
import jax.numpy as jnp
from jax import lax
from jax.experimental import pallas as pl
from jax.experimental.pallas import tpu as pltpu
from jax.experimental.pallas import tpu_sc as plsc

INPUT_DIM = 4096
COLUMN_DIM = 16384
SPARSITY = 0.02
BATCH = 256
K_TOP = int(round(SPARSITY * COLUMN_DIM))
BOOST_BETA = 100.0

BN = 1024
R2 = 64
CAND = 384
RANKW = 384
R4 = 8

_INTERPRET = False


def _matmul_body(x_ref, p_ref, b_ref, ov_ref, bo_ref):
    x = x_ref[...]
    p = p_ref[...]
    ov = jax.lax.dot_general(x, p, (((1,), (1,)), ((), ())),
                             preferred_element_type=jnp.float32)
    ov_ref[...] = ov
    bo_ref[...] = ov * b_ref[...]


def _overlaps_boosted(inp, perm, boost2d):
    return pl.pallas_call(
        _matmul_body,
        grid=(COLUMN_DIM // BN,),
        in_specs=[
            pl.BlockSpec((BATCH, INPUT_DIM), lambda j: (0, 0)),
            pl.BlockSpec((BN, INPUT_DIM), lambda j: (j, 0)),
            pl.BlockSpec((1, BN), lambda j: (0, j)),
        ],
        out_specs=[
            pl.BlockSpec((BATCH, BN), lambda j: (0, j)),
            pl.BlockSpec((BATCH, BN), lambda j: (0, j)),
        ],
        out_shape=[
            jax.ShapeDtypeStruct((BATCH, COLUMN_DIM), jnp.float32),
            jax.ShapeDtypeStruct((BATCH, COLUMN_DIM), jnp.float32),
        ],
        interpret=_INTERPRET,
    )(inp, perm, boost2d)


def _excl_prefix(mask_f32, ltri):
    t3 = mask_f32.reshape(R2 * 128, 128)
    inner = jax.lax.dot_general(t3, ltri, (((1,), (0,)), ((), ())),
                                preferred_element_type=jnp.float32)
    chunk_tot = jnp.sum(t3, axis=1).reshape(R2, 128)
    chunk_off = jax.lax.dot_general(chunk_tot, ltri, (((1,), (0,)), ((), ())),
                                    preferred_element_type=jnp.float32)
    return (inner.reshape(R2, 128, 128)
            + chunk_off[:, :, None]).reshape(R2, COLUMN_DIM)


def _threshold_body(bo_ref, bits_ref, slot_ref):
    v = bo_ref[...]
    vi = jax.lax.bitcast_convert_type(v, jnp.int32)

    def step(_, carry):
        lo, hi = carry
        mid = lo + jax.lax.div(hi - lo + 1, 2)
        cnt = jnp.sum((vi >= mid).astype(jnp.int32), axis=1, keepdims=True)
        pred = cnt >= K_TOP
        return jnp.where(pred, mid, lo), jnp.where(pred, hi, mid - 1)

    lo0 = jnp.zeros((R2, 1), jnp.int32)
    hi0 = jnp.full((R2, 1), int(np.float32(30266.0).view(np.int32)), jnp.int32)
    lo, _ = jax.lax.fori_loop(0, 31, step, (lo0, hi0))
    thr_i = lo
    gt = (vi > thr_i)
    count_gt = jnp.sum(gt.astype(jnp.int32), axis=1, keepdims=True)
    need = K_TOP - count_gt

    ltri = (jax.lax.broadcasted_iota(jnp.int32, (128, 128), 0)
            < jax.lax.broadcasted_iota(jnp.int32, (128, 128), 1)
            ).astype(jnp.float32)
    tie_rank = _excl_prefix((vi == thr_i).astype(jnp.float32), ltri)
    winner = gt | ((vi == thr_i) & (tie_rank < need.astype(jnp.float32)))
    bits_ref[...] = winner.astype(jnp.float32)
    wslot = _excl_prefix(winner.astype(jnp.float32), ltri)
    slot_ref[...] = jnp.where(winner, wslot.astype(jnp.int32), CAND - 1)


def _threshold(boosted):
    return pl.pallas_call(
        _threshold_body,
        grid=(BATCH // R2,),
        in_specs=[pl.BlockSpec((R2, COLUMN_DIM), lambda i: (i, 0))],
        out_specs=[
            pl.BlockSpec((R2, COLUMN_DIM), lambda i: (i, 0)),
            pl.BlockSpec((R2, COLUMN_DIM), lambda i: (i, 0)),
        ],
        out_shape=[
            jax.ShapeDtypeStruct((BATCH, COLUMN_DIM), jnp.float32),
            jax.ShapeDtypeStruct((BATCH, COLUMN_DIM), jnp.int32),
        ],
        interpret=_INTERPRET,
    )(boosted)


_SC_L = 16
_SC_WORKERS = 32


def _sc_compact_body(nrows, slot_hbm, bo_hbm, idx_hbm, val_hbm,
                     slot_v, bo_v, ci_v, cv_v):
    L = _SC_L
    rows_per_w = nrows // _SC_WORKERS
    wid = lax.axis_index("s") * 2 + lax.axis_index("c")
    iota = lax.iota(jnp.int32, L)
    pad_i = jnp.full((L,), COLUMN_DIM, jnp.int32)
    pad_v = jnp.full((L,), -1.0, jnp.float32)
    for rlocal in range(rows_per_w):
        row = wid * rows_per_w + rlocal
        pltpu.sync_copy(slot_hbm.at[row], slot_v)
        pltpu.sync_copy(bo_hbm.at[row], bo_v)
        for j in range(CAND // L):
            ci_v[pl.ds(j * L, L)] = pad_i
            cv_v[pl.ds(j * L, L)] = pad_v

        def body(j, c):
            s = slot_v[pl.ds(j * L, L)]
            v = bo_v[pl.ds(j * L, L)]
            m = s < CAND - 1
            plsc.store_scatter(ci_v, [s], iota + j * L, mask=m)
            plsc.store_scatter(cv_v, [s], v, mask=m)
            return c

        lax.fori_loop(0, COLUMN_DIM // L, body, jnp.int32(0))
        pltpu.sync_copy(ci_v, idx_hbm.at[row])
        pltpu.sync_copy(cv_v, val_hbm.at[row])


def _compact(slot, boosted):
    nrows = slot.shape[0]
    mesh = plsc.VectorSubcoreMesh(core_axis_name="c", subcore_axis_name="s")
    f = pl.kernel(
        functools.partial(_sc_compact_body, nrows), mesh=mesh,
        out_type=[
            jax.ShapeDtypeStruct((nrows, CAND), jnp.int32),
            jax.ShapeDtypeStruct((nrows, CAND), jnp.float32),
        ],
        scratch_types=[
            pltpu.VMEM((COLUMN_DIM,), jnp.int32),
            pltpu.VMEM((COLUMN_DIM,), jnp.float32),
            pltpu.VMEM((CAND,), jnp.int32),
            pltpu.VMEM((CAND,), jnp.float32),
        ],
        compiler_params=pltpu.CompilerParams(needs_layout_passes=False),
    )
    return f(slot, boosted)


def _rank_body(ci_ref, cv_ref, out_ref):
    ci = ci_ref[...]
    cv = cv_ref[...]
    va = cv[:, :, None]
    vb = cv[:, None, :]
    ia = ci[:, :, None]
    ib = ci[:, None, :]
    beats = (vb > va) | ((vb == va) & (ib < ia))
    rank = jnp.sum(beats.astype(jnp.int32), axis=2)
    onehot = (rank[:, :, None]
              == jax.lax.broadcasted_iota(jnp.int32, (1, 1, RANKW), 2)
              ).astype(jnp.float32)
    outf = jax.lax.dot_general(ci.astype(jnp.float32), onehot,
                               (((1,), (1,)), ((0,), (0,))),
                               preferred_element_type=jnp.float32)
    out_ref[...] = outf[:, :K_TOP].astype(jnp.int32)


def _rank_scatter(cand_idx, cand_val):
    nrows = cand_idx.shape[0]
    return pl.pallas_call(
        _rank_body,
        grid=(nrows // R4,),
        in_specs=[
            pl.BlockSpec((R4, CAND), lambda i: (i, 0)),
            pl.BlockSpec((R4, CAND), lambda i: (i, 0)),
        ],
        out_specs=pl.BlockSpec((R4, K_TOP), lambda i: (i, 0)),
        out_shape=jax.ShapeDtypeStruct((nrows, K_TOP), jnp.int32),
        interpret=_INTERPRET,
    )(cand_idx, cand_val)


def kernel(input, permanences, duty_cycles):
    boost = jnp.exp(BOOST_BETA * (SPARSITY - duty_cycles))
    overlaps, boosted = _overlaps_boosted(input, permanences,
                                          boost.reshape(1, COLUMN_DIM))
    bits, slot = _threshold(boosted)
    half = BATCH // 2
    actives = []
    for h in range(2):
        slot_h = jax.lax.slice_in_dim(slot, h * half, (h + 1) * half, axis=0)
        bo_h = jax.lax.slice_in_dim(boosted, h * half, (h + 1) * half, axis=0)
        cand_idx, cand_val = _compact(slot_h, bo_h)
        actives.append(_rank_scatter(cand_idx, cand_val))
    active_columns = jnp.concatenate(actives, axis=0)
    return active_columns, overlaps, boosted, bits


# speedup vs baseline: 5.3132x; 1.0945x over previous
import functools

import numpy as np

import jax
import jax.numpy as jnp
from jax import lax
from jax.experimental import pallas as pl
from jax.experimental.pallas import tpu as pltpu
from jax.experimental.pallas import tpu_sc as plsc

INPUT_DIM = 4096
COLUMN_DIM = 16384
SPARSITY = 0.02
BATCH = 256
K_TOP = int(round(SPARSITY * COLUMN_DIM))
BOOST_BETA = 100.0

BN = 1024
R2 = 64
CAND = 384
RANKW = 384
R4 = 8

_INTERPRET = False


def _matmul_body(x_ref, p_ref, b_ref, ov_ref, bo_ref):
    x = x_ref[...]
    p = p_ref[...]
    ov = jax.lax.dot_general(x, p, (((1,), (1,)), ((), ())),
                             preferred_element_type=jnp.float32)
    ov_ref[...] = ov
    bo_ref[...] = ov * b_ref[...]


def _overlaps_boosted(inp, perm, boost2d):
    return pl.pallas_call(
        _matmul_body,
        grid=(COLUMN_DIM // BN,),
        in_specs=[
            pl.BlockSpec((BATCH, INPUT_DIM), lambda j: (0, 0)),
            pl.BlockSpec((BN, INPUT_DIM), lambda j: (j, 0)),
            pl.BlockSpec((1, BN), lambda j: (0, j)),
        ],
        out_specs=[
            pl.BlockSpec((BATCH, BN), lambda j: (0, j)),
            pl.BlockSpec((BATCH, BN), lambda j: (0, j)),
        ],
        out_shape=[
            jax.ShapeDtypeStruct((BATCH, COLUMN_DIM), jnp.float32),
            jax.ShapeDtypeStruct((BATCH, COLUMN_DIM), jnp.float32),
        ],
        interpret=_INTERPRET,
    )(inp, perm, boost2d)


def _excl_prefix(mask_f32, ltri):
    t3 = mask_f32.reshape(R2 * 128, 128)
    inner = jax.lax.dot_general(t3, ltri, (((1,), (0,)), ((), ())),
                                preferred_element_type=jnp.float32)
    chunk_tot = jnp.sum(t3, axis=1).reshape(R2, 128)
    chunk_off = jax.lax.dot_general(chunk_tot, ltri, (((1,), (0,)), ((), ())),
                                    preferred_element_type=jnp.float32)
    return (inner.reshape(R2, 128, 128)
            + chunk_off[:, :, None]).reshape(R2, COLUMN_DIM)


def _threshold_body(bo_ref, bits_ref, slot_ref):
    v = bo_ref[...]
    vi = jax.lax.bitcast_convert_type(v, jnp.int32)

    def step(_, carry):
        lo, hi = carry
        mid = lo + jax.lax.div(hi - lo + 1, 2)
        cnt = jnp.sum((vi >= mid).astype(jnp.int32), axis=1, keepdims=True)
        pred = cnt >= K_TOP
        return jnp.where(pred, mid, lo), jnp.where(pred, hi, mid - 1)

    lo0 = jnp.zeros((R2, 1), jnp.int32)
    hi0 = jnp.full((R2, 1), int(np.float32(30266.0).view(np.int32)), jnp.int32)
    lo, _ = jax.lax.fori_loop(0, 31, step, (lo0, hi0))
    thr_i = lo
    gt = (vi > thr_i)
    count_gt = jnp.sum(gt.astype(jnp.int32), axis=1, keepdims=True)
    need = K_TOP - count_gt

    ltri = (jax.lax.broadcasted_iota(jnp.int32, (128, 128), 0)
            < jax.lax.broadcasted_iota(jnp.int32, (128, 128), 1)
            ).astype(jnp.float32)
    tie_rank = _excl_prefix((vi == thr_i).astype(jnp.float32), ltri)
    winner = gt | ((vi == thr_i) & (tie_rank < need.astype(jnp.float32)))
    bits_ref[...] = winner.astype(jnp.float32)
    wslot = _excl_prefix(winner.astype(jnp.float32), ltri)
    slot_ref[...] = jnp.where(winner, wslot.astype(jnp.int32), CAND - 1)


def _threshold(boosted):
    return pl.pallas_call(
        _threshold_body,
        grid=(BATCH // R2,),
        in_specs=[pl.BlockSpec((R2, COLUMN_DIM), lambda i: (i, 0))],
        out_specs=[
            pl.BlockSpec((R2, COLUMN_DIM), lambda i: (i, 0)),
            pl.BlockSpec((R2, COLUMN_DIM), lambda i: (i, 0)),
        ],
        out_shape=[
            jax.ShapeDtypeStruct((BATCH, COLUMN_DIM), jnp.float32),
            jax.ShapeDtypeStruct((BATCH, COLUMN_DIM), jnp.int32),
        ],
        interpret=_INTERPRET,
    )(boosted)


_SC_L = 16
_SC_WORKERS = 32


def _sc_compact_body(nrows, slot_hbm, bo_hbm, idx_hbm, val_hbm,
                     slot_v, bo_v, ci_v, cv_v):
    L = _SC_L
    rows_per_w = nrows // _SC_WORKERS
    wid = lax.axis_index("s") * 2 + lax.axis_index("c")
    iota = lax.iota(jnp.int32, L)
    pad_i = jnp.full((L,), COLUMN_DIM, jnp.int32)
    pad_v = jnp.full((L,), -1.0, jnp.float32)
    for rlocal in range(rows_per_w):
        row = wid * rows_per_w + rlocal
        pltpu.sync_copy(slot_hbm.at[row], slot_v)
        pltpu.sync_copy(bo_hbm.at[row], bo_v)
        for j in range(CAND // L):
            ci_v[pl.ds(j * L, L)] = pad_i
            cv_v[pl.ds(j * L, L)] = pad_v

        @plsc.parallel_loop(0, COLUMN_DIM // L, unroll=8)
        def _scatter_iter(j):
            s = slot_v[pl.ds(j * L, L)]
            v = bo_v[pl.ds(j * L, L)]
            m = s < CAND - 1
            plsc.store_scatter(ci_v, [s], iota + j * L, mask=m)
            plsc.store_scatter(cv_v, [s], v, mask=m)
        pltpu.sync_copy(ci_v, idx_hbm.at[row])
        pltpu.sync_copy(cv_v, val_hbm.at[row])


def _compact(slot, boosted):
    nrows = slot.shape[0]
    mesh = plsc.VectorSubcoreMesh(core_axis_name="c", subcore_axis_name="s")
    f = pl.kernel(
        functools.partial(_sc_compact_body, nrows), mesh=mesh,
        out_type=[
            jax.ShapeDtypeStruct((nrows, CAND), jnp.int32),
            jax.ShapeDtypeStruct((nrows, CAND), jnp.float32),
        ],
        scratch_types=[
            pltpu.VMEM((COLUMN_DIM,), jnp.int32),
            pltpu.VMEM((COLUMN_DIM,), jnp.float32),
            pltpu.VMEM((CAND,), jnp.int32),
            pltpu.VMEM((CAND,), jnp.float32),
        ],
        compiler_params=pltpu.CompilerParams(needs_layout_passes=False),
    )
    return f(slot, boosted)


def _rank_body(ci_ref, cv_ref, out_ref):
    ci = ci_ref[...]
    cv = cv_ref[...]
    va = cv[:, :, None]
    vb = cv[:, None, :]
    ia = ci[:, :, None]
    ib = ci[:, None, :]
    beats = (vb > va) | ((vb == va) & (ib < ia))
    rank = jnp.sum(beats.astype(jnp.int32), axis=2)
    iota_w = jax.lax.broadcasted_iota(jnp.int32, (1, 1, RANKW), 2)
    out = jnp.sum(jnp.where(rank[:, :, None] == iota_w, ci[:, :, None], 0),
                  axis=1)
    out_ref[...] = out[:, :K_TOP]


def _rank_scatter(cand_idx, cand_val):
    nrows = cand_idx.shape[0]
    return pl.pallas_call(
        _rank_body,
        grid=(nrows // R4,),
        in_specs=[
            pl.BlockSpec((R4, CAND), lambda i: (i, 0)),
            pl.BlockSpec((R4, CAND), lambda i: (i, 0)),
        ],
        out_specs=pl.BlockSpec((R4, K_TOP), lambda i: (i, 0)),
        out_shape=jax.ShapeDtypeStruct((nrows, K_TOP), jnp.int32),
        interpret=_INTERPRET,
    )(cand_idx, cand_val)


def kernel(input, permanences, duty_cycles):
    boost = jnp.exp(BOOST_BETA * (SPARSITY - duty_cycles))
    overlaps, boosted = _overlaps_boosted(input, permanences,
                                          boost.reshape(1, COLUMN_DIM))
    bits, slot = _threshold(boosted)
    half = BATCH // 2
    actives = []
    for h in range(2):
        slot_h = jax.lax.slice_in_dim(slot, h * half, (h + 1) * half, axis=0)
        bo_h = jax.lax.slice_in_dim(boosted, h * half, (h + 1) * half, axis=0)
        cand_idx, cand_val = _compact(slot_h, bo_h)
        actives.append(_rank_scatter(cand_idx, cand_val))
    active_columns = jnp.concatenate(actives, axis=0)
    return active_columns, overlaps, boosted, bits


# speedup vs baseline: 5.6810x; 1.0692x over previous
import functools

import numpy as np

import jax
import jax.numpy as jnp
from jax import lax
from jax.experimental import pallas as pl
from jax.experimental.pallas import tpu as pltpu
from jax.experimental.pallas import tpu_sc as plsc

INPUT_DIM = 4096
COLUMN_DIM = 16384
SPARSITY = 0.02
BATCH = 256
K_TOP = int(round(SPARSITY * COLUMN_DIM))
BOOST_BETA = 100.0

BN = 1024
R2 = 64
CAND = 384
RANKW = 384
R4 = 8

_INTERPRET = False


def _matmul_body(x_ref, p_ref, b_ref, ov_ref, bo_ref):
    x = x_ref[...]
    p = p_ref[...]
    ov = jax.lax.dot_general(x, p, (((1,), (1,)), ((), ())),
                             preferred_element_type=jnp.float32)
    ov_ref[...] = ov
    bo_ref[...] = ov * b_ref[...]


def _overlaps_boosted(inp, perm, boost2d):
    return pl.pallas_call(
        _matmul_body,
        grid=(COLUMN_DIM // BN,),
        in_specs=[
            pl.BlockSpec((BATCH, INPUT_DIM), lambda j: (0, 0)),
            pl.BlockSpec((BN, INPUT_DIM), lambda j: (j, 0)),
            pl.BlockSpec((1, BN), lambda j: (0, j)),
        ],
        out_specs=[
            pl.BlockSpec((BATCH, BN), lambda j: (0, j)),
            pl.BlockSpec((BATCH, BN), lambda j: (0, j)),
        ],
        out_shape=[
            jax.ShapeDtypeStruct((BATCH, COLUMN_DIM), jnp.float32),
            jax.ShapeDtypeStruct((BATCH, COLUMN_DIM), jnp.float32),
        ],
        interpret=_INTERPRET,
    )(inp, perm, boost2d)


def _excl_prefix(mask_f32, ltri):
    t3 = mask_f32.reshape(R2 * 128, 128)
    inner = jax.lax.dot_general(t3, ltri, (((1,), (0,)), ((), ())),
                                preferred_element_type=jnp.float32)
    chunk_tot = jnp.sum(t3, axis=1).reshape(R2, 128)
    chunk_off = jax.lax.dot_general(chunk_tot, ltri, (((1,), (0,)), ((), ())),
                                    preferred_element_type=jnp.float32)
    return (inner.reshape(R2, 128, 128)
            + chunk_off[:, :, None]).reshape(R2, COLUMN_DIM)


def _threshold_body(bo_ref, bits_ref, slot_ref):
    v = bo_ref[...]
    vi = jax.lax.bitcast_convert_type(v, jnp.int32)

    def step(_, carry):
        lo, hi = carry
        mid = lo + jax.lax.div(hi - lo + 1, 2)
        cnt = jnp.sum((vi >= mid).astype(jnp.int32), axis=1, keepdims=True)
        pred = cnt >= K_TOP
        return jnp.where(pred, mid, lo), jnp.where(pred, hi, mid - 1)

    lo0 = jnp.zeros((R2, 1), jnp.int32)
    hi0 = jnp.full((R2, 1), int(np.float32(30266.0).view(np.int32)), jnp.int32)
    lo, _ = jax.lax.fori_loop(0, 31, step, (lo0, hi0))
    thr_i = lo
    gt = (vi > thr_i)
    count_gt = jnp.sum(gt.astype(jnp.int32), axis=1, keepdims=True)
    need = K_TOP - count_gt

    ltri = (jax.lax.broadcasted_iota(jnp.int32, (128, 128), 0)
            < jax.lax.broadcasted_iota(jnp.int32, (128, 128), 1)
            ).astype(jnp.float32)
    tie_rank = _excl_prefix((vi == thr_i).astype(jnp.float32), ltri)
    winner = gt | ((vi == thr_i) & (tie_rank < need.astype(jnp.float32)))
    bits_ref[...] = winner.astype(jnp.float32)
    wslot = _excl_prefix(winner.astype(jnp.float32), ltri)
    slot_ref[...] = jnp.where(winner, wslot.astype(jnp.int32), CAND - 1)


def _threshold(boosted):
    return pl.pallas_call(
        _threshold_body,
        grid=(BATCH // R2,),
        in_specs=[pl.BlockSpec((R2, COLUMN_DIM), lambda i: (i, 0))],
        out_specs=[
            pl.BlockSpec((R2, COLUMN_DIM), lambda i: (i, 0)),
            pl.BlockSpec((R2, COLUMN_DIM), lambda i: (i, 0)),
        ],
        out_shape=[
            jax.ShapeDtypeStruct((BATCH, COLUMN_DIM), jnp.float32),
            jax.ShapeDtypeStruct((BATCH, COLUMN_DIM), jnp.int32),
        ],
        interpret=_INTERPRET,
    )(boosted)


_SC_L = 16
_SC_WORKERS = 32


def _sc_compact_body(row0, nrows, slot_hbm, bo_hbm, idx_hbm, val_hbm,
                     slot_v, bo_v, ci_v, cv_v):
    L = _SC_L
    rows_per_w = nrows // _SC_WORKERS
    wid = lax.axis_index("s") * 2 + lax.axis_index("c")
    iota = lax.iota(jnp.int32, L)
    pad_i = jnp.full((L,), COLUMN_DIM, jnp.int32)
    pad_v = jnp.full((L,), -1.0, jnp.float32)
    for rlocal in range(rows_per_w):
        row = wid * rows_per_w + rlocal
        pltpu.sync_copy(slot_hbm.at[row0 + row], slot_v)
        pltpu.sync_copy(bo_hbm.at[row0 + row], bo_v)
        for j in range(CAND // L):
            ci_v[pl.ds(j * L, L)] = pad_i
            cv_v[pl.ds(j * L, L)] = pad_v

        @plsc.parallel_loop(0, COLUMN_DIM // L, unroll=8)
        def _scatter_iter(j):
            s = slot_v[pl.ds(j * L, L)]
            v = bo_v[pl.ds(j * L, L)]
            m = s < CAND - 1
            plsc.store_scatter(ci_v, [s], iota + j * L, mask=m)
            plsc.store_scatter(cv_v, [s], v, mask=m)
        pltpu.sync_copy(ci_v, idx_hbm.at[row])
        pltpu.sync_copy(cv_v, val_hbm.at[row])


def _compact(slot, boosted, row0, nrows):
    mesh = plsc.VectorSubcoreMesh(core_axis_name="c", subcore_axis_name="s")
    f = pl.kernel(
        functools.partial(_sc_compact_body, row0, nrows), mesh=mesh,
        out_type=[
            jax.ShapeDtypeStruct((nrows, CAND), jnp.int32),
            jax.ShapeDtypeStruct((nrows, CAND), jnp.float32),
        ],
        scratch_types=[
            pltpu.VMEM((COLUMN_DIM,), jnp.int32),
            pltpu.VMEM((COLUMN_DIM,), jnp.float32),
            pltpu.VMEM((CAND,), jnp.int32),
            pltpu.VMEM((CAND,), jnp.float32),
        ],
        compiler_params=pltpu.CompilerParams(needs_layout_passes=False),
    )
    return f(slot, boosted)


def _rank_body(ci_ref, cv_ref, out_ref):
    ci = ci_ref[...]
    cv = cv_ref[...]
    va = cv[:, :, None]
    vb = cv[:, None, :]
    ia = ci[:, :, None]
    ib = ci[:, None, :]
    beats = (vb > va) | ((vb == va) & (ib < ia))
    rank = jnp.sum(beats.astype(jnp.int32), axis=2)
    iota_w = jax.lax.broadcasted_iota(jnp.int32, (1, 1, RANKW), 2)
    out = jnp.sum(jnp.where(rank[:, :, None] == iota_w, ci[:, :, None], 0),
                  axis=1)
    out_ref[...] = out[:, :K_TOP]


def _rank_scatter(cand_idx, cand_val):
    nrows = cand_idx.shape[0]
    return pl.pallas_call(
        _rank_body,
        grid=(nrows // R4,),
        in_specs=[
            pl.BlockSpec((R4, CAND), lambda i: (i, 0)),
            pl.BlockSpec((R4, CAND), lambda i: (i, 0)),
        ],
        out_specs=pl.BlockSpec((R4, K_TOP), lambda i: (i, 0)),
        out_shape=jax.ShapeDtypeStruct((nrows, K_TOP), jnp.int32),
        interpret=_INTERPRET,
    )(cand_idx, cand_val)


def kernel(input, permanences, duty_cycles):
    boost = jnp.exp(BOOST_BETA * (SPARSITY - duty_cycles))
    overlaps, boosted = _overlaps_boosted(input, permanences,
                                          boost.reshape(1, COLUMN_DIM))
    bits, slot = _threshold(boosted)
    half = BATCH // 2
    actives = []
    for h in range(2):
        cand_idx, cand_val = _compact(slot, boosted, h * half, half)
        actives.append(_rank_scatter(cand_idx, cand_val))
    active_columns = jnp.concatenate(actives, axis=0)
    return active_columns, overlaps, boosted, bits


# speedup vs baseline: 5.6905x; 1.0017x over previous
import functools

import numpy as np

import jax
import jax.numpy as jnp
from jax import lax
from jax.experimental import pallas as pl
from jax.experimental.pallas import tpu as pltpu
from jax.experimental.pallas import tpu_sc as plsc

INPUT_DIM = 4096
COLUMN_DIM = 16384
SPARSITY = 0.02
BATCH = 256
K_TOP = int(round(SPARSITY * COLUMN_DIM))
BOOST_BETA = 100.0

BN = 1024
R2 = 64
CAND = 384
RANKW = 384
R4 = 16

_INTERPRET = False


def _matmul_body(x_ref, p_ref, b_ref, ov_ref, bo_ref):
    x = x_ref[...]
    p = p_ref[...]
    ov = jax.lax.dot_general(x, p, (((1,), (1,)), ((), ())),
                             preferred_element_type=jnp.float32)
    ov_ref[...] = ov
    bo_ref[...] = ov * b_ref[...]


def _overlaps_boosted(inp, perm, boost2d):
    return pl.pallas_call(
        _matmul_body,
        grid=(COLUMN_DIM // BN,),
        in_specs=[
            pl.BlockSpec((BATCH, INPUT_DIM), lambda j: (0, 0)),
            pl.BlockSpec((BN, INPUT_DIM), lambda j: (j, 0)),
            pl.BlockSpec((1, BN), lambda j: (0, j)),
        ],
        out_specs=[
            pl.BlockSpec((BATCH, BN), lambda j: (0, j)),
            pl.BlockSpec((BATCH, BN), lambda j: (0, j)),
        ],
        out_shape=[
            jax.ShapeDtypeStruct((BATCH, COLUMN_DIM), jnp.float32),
            jax.ShapeDtypeStruct((BATCH, COLUMN_DIM), jnp.float32),
        ],
        interpret=_INTERPRET,
    )(inp, perm, boost2d)


def _excl_prefix(mask_f32, ltri):
    t3 = mask_f32.reshape(R2 * 128, 128)
    inner = jax.lax.dot_general(t3, ltri, (((1,), (0,)), ((), ())),
                                preferred_element_type=jnp.float32)
    chunk_tot = jnp.sum(t3, axis=1).reshape(R2, 128)
    chunk_off = jax.lax.dot_general(chunk_tot, ltri, (((1,), (0,)), ((), ())),
                                    preferred_element_type=jnp.float32)
    return (inner.reshape(R2, 128, 128)
            + chunk_off[:, :, None]).reshape(R2, COLUMN_DIM)


def _threshold_body(bo_ref, bits_ref, slot_ref):
    v = bo_ref[...]
    vi = jax.lax.bitcast_convert_type(v, jnp.int32)

    def step(_, carry):
        lo, hi = carry
        mid = lo + jax.lax.div(hi - lo + 1, 2)
        cnt = jnp.sum((vi >= mid).astype(jnp.int32), axis=1, keepdims=True)
        pred = cnt >= K_TOP
        return jnp.where(pred, mid, lo), jnp.where(pred, hi, mid - 1)

    lo0 = jnp.zeros((R2, 1), jnp.int32)
    hi0 = jnp.full((R2, 1), int(np.float32(30266.0).view(np.int32)), jnp.int32)
    lo, _ = jax.lax.fori_loop(0, 31, step, (lo0, hi0))
    thr_i = lo
    gt = (vi > thr_i)
    count_gt = jnp.sum(gt.astype(jnp.int32), axis=1, keepdims=True)
    need = K_TOP - count_gt

    ltri = (jax.lax.broadcasted_iota(jnp.int32, (128, 128), 0)
            < jax.lax.broadcasted_iota(jnp.int32, (128, 128), 1)
            ).astype(jnp.float32)
    tie_rank = _excl_prefix((vi == thr_i).astype(jnp.float32), ltri)
    winner = gt | ((vi == thr_i) & (tie_rank < need.astype(jnp.float32)))
    bits_ref[...] = winner.astype(jnp.float32)
    wslot = _excl_prefix(winner.astype(jnp.float32), ltri)
    slot_ref[...] = jnp.where(winner, wslot.astype(jnp.int32), CAND - 1)


def _threshold(boosted):
    return pl.pallas_call(
        _threshold_body,
        grid=(BATCH // R2,),
        in_specs=[pl.BlockSpec((R2, COLUMN_DIM), lambda i: (i, 0))],
        out_specs=[
            pl.BlockSpec((R2, COLUMN_DIM), lambda i: (i, 0)),
            pl.BlockSpec((R2, COLUMN_DIM), lambda i: (i, 0)),
        ],
        out_shape=[
            jax.ShapeDtypeStruct((BATCH, COLUMN_DIM), jnp.float32),
            jax.ShapeDtypeStruct((BATCH, COLUMN_DIM), jnp.int32),
        ],
        interpret=_INTERPRET,
    )(boosted)


_SC_L = 16
_SC_WORKERS = 32


def _sc_compact_body(row0, nrows, slot_hbm, bo_hbm, idx_hbm, val_hbm,
                     slot_v, bo_v, ci_v, cv_v):
    L = _SC_L
    rows_per_w = nrows // _SC_WORKERS
    wid = lax.axis_index("s") * 2 + lax.axis_index("c")
    iota = lax.iota(jnp.int32, L)
    pad_i = jnp.full((L,), COLUMN_DIM, jnp.int32)
    pad_v = jnp.full((L,), -1.0, jnp.float32)
    for rlocal in range(rows_per_w):
        row = wid * rows_per_w + rlocal
        pltpu.sync_copy(slot_hbm.at[row0 + row], slot_v)
        pltpu.sync_copy(bo_hbm.at[row0 + row], bo_v)
        for j in range(CAND // L):
            ci_v[pl.ds(j * L, L)] = pad_i
            cv_v[pl.ds(j * L, L)] = pad_v

        @plsc.parallel_loop(0, COLUMN_DIM // L, unroll=8)
        def _scatter_iter(j):
            s = slot_v[pl.ds(j * L, L)]
            v = bo_v[pl.ds(j * L, L)]
            m = s < CAND - 1
            plsc.store_scatter(ci_v, [s], iota + j * L, mask=m)
            plsc.store_scatter(cv_v, [s], v, mask=m)
        pltpu.sync_copy(ci_v, idx_hbm.at[row])
        pltpu.sync_copy(cv_v, val_hbm.at[row])


def _compact(slot, boosted, row0, nrows):
    mesh = plsc.VectorSubcoreMesh(core_axis_name="c", subcore_axis_name="s")
    f = pl.kernel(
        functools.partial(_sc_compact_body, row0, nrows), mesh=mesh,
        out_type=[
            jax.ShapeDtypeStruct((nrows, CAND), jnp.int32),
            jax.ShapeDtypeStruct((nrows, CAND), jnp.float32),
        ],
        scratch_types=[
            pltpu.VMEM((COLUMN_DIM,), jnp.int32),
            pltpu.VMEM((COLUMN_DIM,), jnp.float32),
            pltpu.VMEM((CAND,), jnp.int32),
            pltpu.VMEM((CAND,), jnp.float32),
        ],
        compiler_params=pltpu.CompilerParams(needs_layout_passes=False),
    )
    return f(slot, boosted)


def _rank_body(ci_ref, cv_ref, out_ref):
    ci = ci_ref[...]
    cv = cv_ref[...]
    va = cv[:, :, None]
    vb = cv[:, None, :]
    ia = ci[:, :, None]
    ib = ci[:, None, :]
    beats = (vb > va) | ((vb == va) & (ib < ia))
    rank = jnp.sum(beats.astype(jnp.int32), axis=2)
    iota_w = jax.lax.broadcasted_iota(jnp.int32, (1, 1, RANKW), 2)
    out = jnp.sum(jnp.where(rank[:, :, None] == iota_w, ci[:, :, None], 0),
                  axis=1)
    out_ref[...] = out[:, :K_TOP]


def _rank_scatter(cand_idx, cand_val):
    nrows = cand_idx.shape[0]
    return pl.pallas_call(
        _rank_body,
        grid=(nrows // R4,),
        in_specs=[
            pl.BlockSpec((R4, CAND), lambda i: (i, 0)),
            pl.BlockSpec((R4, CAND), lambda i: (i, 0)),
        ],
        out_specs=pl.BlockSpec((R4, K_TOP), lambda i: (i, 0)),
        out_shape=jax.ShapeDtypeStruct((nrows, K_TOP), jnp.int32),
        interpret=_INTERPRET,
    )(cand_idx, cand_val)


def kernel(input, permanences, duty_cycles):
    boost = jnp.exp(BOOST_BETA * (SPARSITY - duty_cycles))
    overlaps, boosted = _overlaps_boosted(input, permanences,
                                          boost.reshape(1, COLUMN_DIM))
    bits, slot = _threshold(boosted)
    half = BATCH // 2
    actives = []
    for h in range(2):
        cand_idx, cand_val = _compact(slot, boosted, h * half, half)
        actives.append(_rank_scatter(cand_idx, cand_val))
    active_columns = jnp.concatenate(actives, axis=0)
    return active_columns, overlaps, boosted, bits
